# Initial kernel scaffold; baseline (speedup 1.0000x reference)
import jax
import jax.numpy as jnp
from jax.experimental import pallas as pl


def kernel(x, attn_norm_w, ffn_norm_w, qkv_w, qkv_b, proj_w, proj_b, attn_sink, gate_w, gate_b, up_w, down_w):
    raise NotImplementedError("write your pallas kernel here")


# speedup vs baseline: 1.2602x; 1.2602x over previous
import functools
import math

import jax
import jax.numpy as jnp
from jax.experimental import pallas as pl
from jax.experimental.pallas import tpu as pltpu

D_MODEL = 1024
N_HEADS = 16
N_KV = 8
DH = 64
D_FF = 1024
N_EXP = 8
TOP_K = 2
EPS = 1e-05
ROPE_BASE = 150000.0
ROPE_SCALE = 32.0
T = 2048
QKV_DIM = N_HEADS * DH + N_KV * DH + N_KV * DH

F32 = jnp.float32
BF16 = jnp.bfloat16


def _qkv_body(x_ref, nw_ref, w_ref, b_ref, sin_ref, cos_ref,
              q_ref, k_ref, v_ref):
    x = x_ref[...]
    ms = jnp.mean(x * x, axis=1, keepdims=True)
    xn = (x * jax.lax.rsqrt(ms + EPS)) * nw_ref[...]
    qkv = jnp.dot(xn.astype(BF16), w_ref[...],
                  preferred_element_type=F32) + b_ref[...]
    sin = sin_ref[...]
    cos = cos_ref[...]

    def rope(h):
        h1 = h[:, :DH // 2]
        h2 = h[:, DH // 2:]
        return jnp.concatenate([h1 * cos - h2 * sin, h1 * sin + h2 * cos],
                               axis=1)

    for h in range(N_HEADS):
        q_ref[h] = rope(qkv[:, h * DH:(h + 1) * DH]).astype(BF16)
    koff = N_HEADS * DH
    voff = koff + N_KV * DH
    for h in range(N_KV):
        k_ref[h] = rope(qkv[:, koff + h * DH:koff + (h + 1) * DH]).astype(BF16)
        v_ref[h] = qkv[:, voff + h * DH:voff + (h + 1) * DH].astype(BF16)


def _qkv_call(x2d, nw, w_bf, b, sin, cos):
    return pl.pallas_call(
        _qkv_body,
        out_shape=(
            jax.ShapeDtypeStruct((N_HEADS, T, DH), BF16),
            jax.ShapeDtypeStruct((N_KV, T, DH), BF16),
            jax.ShapeDtypeStruct((N_KV, T, DH), BF16),
        ),
    )(x2d, nw, w_bf, b, sin, cos)


TQ = 512


def _attn_body(q_ref, k_ref, v_ref, sink_ref, y_ref):
    qi = pl.program_id(1)
    q = q_ref[0]
    k = k_ref[0]
    v = v_ref[0]
    logits = jax.lax.dot_general(
        q, k, (((1,), (1,)), ((), ())), preferred_element_type=F32) * 0.125
    row = qi * TQ + jax.lax.broadcasted_iota(jnp.int32, (TQ, T), 0)
    col = jax.lax.broadcasted_iota(jnp.int32, (TQ, T), 1)
    logits = jnp.where(col > row, -jnp.inf, logits)
    sink = sink_ref[0, 0, 0]
    m = jnp.maximum(jnp.max(logits, axis=1, keepdims=True), sink)
    p = jnp.exp(logits - m)
    denom = jnp.sum(p, axis=1, keepdims=True) + jnp.exp(sink - m)
    y = jax.lax.dot_general(p.astype(BF16), v, (((1,), (0,)), ((), ())),
                            preferred_element_type=F32)
    y_ref[:, 0, 0, :] = (y / denom).astype(BF16)


def _attn_call(q, k, v, sink):
    return pl.pallas_call(
        _attn_body,
        grid=(N_HEADS, T // TQ),
        in_specs=[
            pl.BlockSpec((1, TQ, DH), lambda h, i: (h, i, 0)),
            pl.BlockSpec((1, T, DH), lambda h, i: (h // 2, 0, 0)),
            pl.BlockSpec((1, T, DH), lambda h, i: (h // 2, 0, 0)),
            pl.BlockSpec((1, 1, 1), lambda h, i: (h, 0, 0)),
        ],
        out_specs=pl.BlockSpec((TQ, 1, 1, DH), lambda h, i: (i, h, 0, 0)),
        out_shape=jax.ShapeDtypeStruct((T, N_HEADS, 1, DH), BF16),
    )(q, k, v, sink)


def _proj_router_body(x_ref, y_ref, pw_ref, pb_ref, fw_ref, gw_ref, gb_ref,
                      xa_ref, x2_ref, probs_ref):
    proj = jnp.dot(y_ref[...], pw_ref[...],
                   preferred_element_type=F32) + pb_ref[...]
    xa = x_ref[...] + proj
    xa_ref[...] = xa
    ms = jnp.mean(xa * xa, axis=1, keepdims=True)
    x2 = (xa * jax.lax.rsqrt(ms + EPS)) * fw_ref[...]
    x2b = x2.astype(BF16)
    x2_ref[...] = x2b
    gl = jnp.dot(x2b, gw_ref[...], preferred_element_type=F32) + gb_ref[...]
    lane = jax.lax.broadcasted_iota(jnp.int32, (T, 128), 1)
    gl = jnp.where(lane < N_EXP, gl, -jnp.inf)
    v1 = jnp.max(gl, axis=1, keepdims=True)
    i1 = jnp.min(jnp.where(gl == v1, lane, 128), axis=1, keepdims=True)
    gl2 = jnp.where(lane == i1, -jnp.inf, gl)
    v2 = jnp.max(gl2, axis=1, keepdims=True)
    i2 = jnp.min(jnp.where(gl2 == v2, lane, 128), axis=1, keepdims=True)
    w1 = jax.nn.sigmoid(v1 - v2)
    w2 = jax.nn.sigmoid(v2 - v1)
    for e in range(N_EXP):
        pe = jnp.where(i1 == e, w1, 0.0) + jnp.where(i2 == e, w2, 0.0)
        probs_ref[e] = pe


def _proj_router_call(x2d, y2d, pw_bf, pb, fw, gw_bf, gb):
    return pl.pallas_call(
        _proj_router_body,
        out_shape=(
            jax.ShapeDtypeStruct((T, D_MODEL), F32),
            jax.ShapeDtypeStruct((T, D_MODEL), BF16),
            jax.ShapeDtypeStruct((N_EXP, T, 1), F32),
        ),
    )(x2d, y2d, pw_bf, pb, fw, gw_bf, gb)


TM = 1024


def _moe_body(x2_ref, up_ref, dn_ref, probs_ref, xa_ref, out_ref):
    e = pl.program_id(1)
    h = jnp.dot(x2_ref[...], up_ref[0], preferred_element_type=F32)
    u = h[:, :D_FF]
    g = h[:, D_FF:]
    act = (jax.nn.silu(g) * u).astype(BF16)
    ye = jnp.dot(act, dn_ref[0], preferred_element_type=F32)
    ce = probs_ref[0]
    contrib = ce * ye

    @pl.when(e == 0)
    def _():
        out_ref[...] = xa_ref[...] + contrib

    @pl.when(e > 0)
    def _():
        out_ref[...] += contrib


def _moe_call(x2b, up_bf, dn_bf, probs, xa):
    return pl.pallas_call(
        _moe_body,
        grid=(T // TM, N_EXP),
        in_specs=[
            pl.BlockSpec((TM, D_MODEL), lambda t, e: (t, 0)),
            pl.BlockSpec((1, D_MODEL, 2 * D_FF), lambda t, e: (e, 0, 0)),
            pl.BlockSpec((1, D_FF, D_MODEL), lambda t, e: (e, 0, 0)),
            pl.BlockSpec((1, TM, 1), lambda t, e: (e, t, 0)),
            pl.BlockSpec((TM, D_MODEL), lambda t, e: (t, 0)),
        ],
        out_specs=pl.BlockSpec((TM, D_MODEL), lambda t, e: (t, 0)),
        out_shape=jax.ShapeDtypeStruct((T, D_MODEL), F32),
    )(x2b, up_bf, dn_bf, probs, xa)


def _rope_tables():
    pos = jnp.arange(T, dtype=F32) / ROPE_SCALE
    idx = jnp.arange(0, DH, 2, dtype=F32)
    inv_freq = 1.0 / (ROPE_BASE ** (idx / DH))
    freqs = jnp.einsum('t,f->tf', pos, inv_freq)
    return jnp.sin(freqs), jnp.cos(freqs)


def _deinterleave_qkv_w(qkv_w, qkv_b):
    half = jnp.arange(0, DH, 2)
    q_idx = jnp.concatenate([jnp.concatenate([h * DH + half, h * DH + half + 1])
                             for h in range(N_HEADS)])
    koff = N_HEADS * DH
    k_idx = jnp.concatenate([koff + jnp.concatenate([h * DH + half,
                                                     h * DH + half + 1])
                             for h in range(N_KV)])
    voff = koff + N_KV * DH
    v_idx = voff + jnp.arange(N_KV * DH)
    col = jnp.concatenate([q_idx, k_idx, v_idx])
    return qkv_w[:, col], qkv_b[col]


def kernel(x, attn_norm_w, ffn_norm_w, qkv_w, qkv_b, proj_w, proj_b,
           attn_sink, gate_w, gate_b, up_w, down_w):
    B = x.shape[0]
    x2d = x.reshape(T, D_MODEL)
    sin, cos = _rope_tables()
    w_perm, b_perm = _deinterleave_qkv_w(qkv_w, qkv_b)

    q, k, v = _qkv_call(
        x2d, attn_norm_w.reshape(1, D_MODEL), w_perm.astype(BF16),
        b_perm.reshape(1, QKV_DIM), sin, cos)

    y = _attn_call(q, k, v, attn_sink.reshape(N_HEADS, 1, 1))
    y2d = y.reshape(T, N_HEADS * DH)

    gw_pad = jnp.zeros((D_MODEL, 128), F32).at[:, :N_EXP].set(gate_w)
    gb_pad = jnp.zeros((1, 128), F32).at[0, :N_EXP].set(gate_b)
    xa, x2b, probs = _proj_router_call(
        x2d, y2d, proj_w.astype(BF16), proj_b.reshape(1, D_MODEL),
        ffn_norm_w.reshape(1, D_MODEL), gw_pad.astype(BF16), gb_pad)

    out = _moe_call(x2b, up_w.astype(BF16), down_w.astype(BF16), probs, xa)
    return out.reshape(B, T, D_MODEL)
